# baseline (speedup 1.0000x reference)
import functools

import jax
import jax.numpy as jnp
from jax import lax
from jax.experimental import pallas as pl
from jax.experimental.pallas import tpu as pltpu
from jax.experimental.pallas import tpu_sc as plsc

_NUM_CORES = 2
_NUM_SUBCORES = 16
_NW = _NUM_CORES * _NUM_SUBCORES
_L = 16


def _sc_body(B, D, P, K, C, x_hbm, thT_hbm, lk_hbm, i_hbm, j_hbm, out_hbm,
             x_chunk, i_buf, j_buf, kthT, out_buf, lk_buf):
    GB = 8
    rows_per_w = B // _NW
    n_chunks = rows_per_w // C
    n_gblks = P // (_L * GB)

    wid = lax.axis_index("s") * _NUM_CORES + lax.axis_index("c")

    pltpu.sync_copy(i_hbm, i_buf)
    pltpu.sync_copy(j_hbm, j_buf)
    pltpu.sync_copy(thT_hbm, kthT)
    pltpu.sync_copy(lk_hbm, lk_buf)

    kappa = jnp.clip(jnp.exp(lk_buf[...]), 0.5, 50.0)

    iota = lax.iota(jnp.int32, _L)
    col_idx = [iota * K + k for k in range(K)]

    def chunk_body(c, _):
        row_base = wid * rows_per_w + c * C
        pltpu.sync_copy(x_hbm.at[pl.ds(row_base, C), :], x_chunk)

        def gblk_body(gb, _):
            for sg in range(GB):
                g = gb * GB + sg
                gi = i_buf[pl.ds(g * _L, _L)]
                gj = j_buf[pl.ds(g * _L, _L)]
                kth = [kappa * kthT[k, pl.ds(g * _L, _L)] for k in range(K)]
                cols = [col_idx[k] + sg * _L * K for k in range(K)]

                def row_body(r, _, gi=gi, gj=gj, kth=kth, cols=cols):
                    rsp = jnp.full((_L,), r, jnp.int32)
                    xi = plsc.load_gather(x_chunk, [rsp, gi])
                    xj = plsc.load_gather(x_chunk, [rsp, gj])
                    kd = kappa * (xi - xj)
                    for k in range(K):
                        z = kd - kth[k]
                        s = 1.0 / (1.0 + jnp.exp(-z))
                        plsc.store_scatter(out_buf, [rsp, cols[k]], s)
                    return _
                lax.fori_loop(0, C, row_body, None)
            pltpu.sync_copy(
                out_buf,
                out_hbm.at[pl.ds(row_base, C), pl.ds(gb * GB * _L * K, GB * _L * K)])
            return _
        lax.fori_loop(0, n_gblks, gblk_body, None)
        return _
    lax.fori_loop(0, n_chunks, chunk_body, None)


def kernel(x, th, log_kappa, i_idx, j_idx):
    B, D = x.shape
    P, K = th.shape
    C = 64

    thT = th.T.reshape(K, P)
    i32 = i_idx.astype(jnp.int32)
    j32 = j_idx.astype(jnp.int32)
    lk = jnp.full((_L,), log_kappa, jnp.float32)

    mesh = plsc.VectorSubcoreMesh(
        core_axis_name="c", subcore_axis_name="s",
        num_cores=_NUM_CORES, num_subcores=_NUM_SUBCORES)
    f = pl.kernel(
        functools.partial(_sc_body, B, D, P, K, C),
        out_type=jax.ShapeDtypeStruct((B, P * K), jnp.float32),
        mesh=mesh,
        compiler_params=pltpu.CompilerParams(needs_layout_passes=False),
        scratch_types=[
            pltpu.VMEM((C, D), jnp.float32),
            pltpu.VMEM((P,), jnp.int32),
            pltpu.VMEM((P,), jnp.int32),
            pltpu.VMEM((K, P), jnp.float32),
            pltpu.VMEM((C, 8 * _L * K), jnp.float32),
            pltpu.VMEM((_L,), jnp.float32),
        ],
    )
    return f(x, thT, lk, i32, j32)
